# baseline (speedup 1.0000x reference)
import functools

import jax
import jax.numpy as jnp
from jax import lax
from jax.experimental import pallas as pl
from jax.experimental.pallas import tpu as pltpu
from jax.experimental.pallas import tpu_sc as plsc

B = 16384
D = 64
C = 100000
SCALE = 0.05

NC = 2
NS = 16
SPT = B // NS
NCH = SPT // 128
RANGE = 25600
PASSES = 2
CHUNK = RANGE // NS
TAIL = C - (2 * PASSES - 1) * RANGE - (NS - 2) * CHUNK
TRASH = RANGE


def _body(feat_hbm, lab_hbm, ctr_hbm, out_hbm, loss_hbm,
          f_v, d_v, lab_v, idx_v, part_v, acc_sh, sem):
    cid = lax.axis_index("c")
    sid = lax.axis_index("s")
    s0 = sid * SPT

    for j in range(NCH):
        pltpu.sync_copy(lab_hbm.at[pl.ds(s0 + j * 128, 128)], lab_v.at[j])

    sq = jnp.zeros((16,), jnp.float32)

    for p in range(PASSES):
        base = (cid * PASSES + p) * RANGE
        row0 = base + sid * CHUNK
        full = row0 + CHUNK <= C
        tail = jnp.logical_and(row0 < C, jnp.logical_not(full))

        @pl.when(full)
        def _(row0=row0):
            pltpu.sync_copy(ctr_hbm.at[pl.ds(row0, CHUNK)],
                            acc_sh.at[pl.ds(sid * CHUNK, CHUNK)])

        @pl.when(tail)
        def _(row0=row0):
            pltpu.sync_copy(ctr_hbm.at[pl.ds(row0, TAIL)],
                            acc_sh.at[pl.ds(sid * CHUNK, TAIL)])

        for j in range(NCH):
            for k in range(8):
                v = lab_v[j, pl.ds(k * 16, 16)]
                m = jnp.logical_and(v >= base, v < base + RANGE)
                idx_v[j, pl.ds(k * 16, 16)] = jnp.where(m, v - base, TRASH)

        plsc.subcore_barrier()

        for j in range(NCH):
            pltpu.sync_copy(feat_hbm.at[pl.ds(s0 + j * 128, 128)], f_v)
            pltpu.async_copy(ctr_hbm.at[lab_v.at[j]], d_v, sem).wait()

            if p == 0:
                def row_body(r, sq):
                    for k in range(4):
                        f = f_v[r, pl.ds(k * 16, 16)]
                        c = d_v[r, pl.ds(k * 16, 16)]
                        d = f - c
                        sq = sq + d * d
                        d_v[r, pl.ds(k * 16, 16)] = d * SCALE
                    return sq
                sq = lax.fori_loop(0, 128, row_body, sq)
            else:
                def row_body(r, carry):
                    for k in range(4):
                        f = f_v[r, pl.ds(k * 16, 16)]
                        c = d_v[r, pl.ds(k * 16, 16)]
                        d_v[r, pl.ds(k * 16, 16)] = (f - c) * SCALE
                    return carry
                lax.fori_loop(0, 128, row_body, 0)

            pltpu.async_copy(d_v, acc_sh.at[idx_v.at[j]], sem, add=True).wait()

        plsc.subcore_barrier()

        @pl.when(full)
        def _(row0=row0):
            pltpu.sync_copy(acc_sh.at[pl.ds(sid * CHUNK, CHUNK)],
                            out_hbm.at[pl.ds(row0, CHUNK)])

        @pl.when(tail)
        def _(row0=row0):
            pltpu.sync_copy(acc_sh.at[pl.ds(sid * CHUNK, TAIL)],
                            out_hbm.at[pl.ds(row0, TAIL)])

        plsc.subcore_barrier()

    part_v[...] = sq

    @pl.when(cid == 0)
    def _():
        pltpu.sync_copy(part_v, loss_hbm.at[sid])


_sc_call = functools.partial(
    pl.kernel,
    out_type=(jax.ShapeDtypeStruct((C, D), jnp.float32),
              jax.ShapeDtypeStruct((NS, 16), jnp.float32)),
    mesh=plsc.VectorSubcoreMesh(core_axis_name="c", subcore_axis_name="s",
                                num_cores=NC, num_subcores=NS),
    scratch_types=[
        pltpu.VMEM((128, D), jnp.float32),
        pltpu.VMEM((128, D), jnp.float32),
        pltpu.VMEM((NCH, 128), jnp.int32),
        pltpu.VMEM((NCH, 128), jnp.int32),
        pltpu.VMEM((16,), jnp.float32),
        pltpu.VMEM_SHARED((RANGE + 8, D), jnp.float32),
        pltpu.SemaphoreType.DMA,
    ],
    compiler_params=pltpu.CompilerParams(use_tc_tiling_on_sc=False),
)(_body)


def kernel(features, labels, center_var):
    labels = labels.reshape(-1)
    new_center, parts = _sc_call(features, labels, center_var)
    loss = jnp.sum(parts) * (1.0 / (B * D))
    return loss, new_center


# speedup vs baseline: 1.0155x; 1.0155x over previous
import functools

import jax
import jax.numpy as jnp
from jax import lax
from jax.experimental import pallas as pl
from jax.experimental.pallas import tpu as pltpu
from jax.experimental.pallas import tpu_sc as plsc

B = 16384
D = 64
C = 100000
SCALE = 0.05

NC = 2
NS = 16
SPT = B // NS
NCH = SPT // 128
RANGE = 25600
PASSES = 2
CHUNK = RANGE // NS
TAIL = C - (2 * PASSES - 1) * RANGE - (NS - 2) * CHUNK
NTRASH = 1024


def _body(feat_hbm, lab_hbm, ctr_hbm, out_hbm, loss_hbm,
          f_v, d_v, lab_v, idx_v, part_v, acc_sh, sem):
    cid = lax.axis_index("c")
    sid = lax.axis_index("s")
    s0 = sid * SPT

    for j in range(NCH):
        pltpu.sync_copy(lab_hbm.at[pl.ds(s0 + j * 128, 128)], lab_v.at[j])

    sq = jnp.zeros((16,), jnp.float32)

    for p in range(PASSES):
        base = (cid * PASSES + p) * RANGE
        row0 = base + sid * CHUNK
        full = row0 + CHUNK <= C
        tail = jnp.logical_and(row0 < C, jnp.logical_not(full))

        @pl.when(full)
        def _(row0=row0):
            pltpu.sync_copy(ctr_hbm.at[pl.ds(row0, CHUNK)],
                            acc_sh.at[pl.ds(sid * CHUNK, CHUNK)])

        @pl.when(tail)
        def _(row0=row0):
            pltpu.sync_copy(ctr_hbm.at[pl.ds(row0, TAIL)],
                            acc_sh.at[pl.ds(sid * CHUNK, TAIL)])

        for j in range(NCH):
            for k in range(8):
                v = lab_v[j, pl.ds(k * 16, 16)]
                m = jnp.logical_and(v >= base, v < base + RANGE)
                trash = RANGE + jnp.bitwise_and(v, NTRASH - 1)
                idx_v[j, pl.ds(k * 16, 16)] = jnp.where(m, v - base, trash)

        plsc.subcore_barrier()

        for j in range(NCH):
            pltpu.sync_copy(feat_hbm.at[pl.ds(s0 + j * 128, 128)], f_v)
            pltpu.async_copy(ctr_hbm.at[lab_v.at[j]], d_v, sem).wait()

            if p == 0:
                def row_body(r, sq):
                    for k in range(4):
                        f = f_v[r, pl.ds(k * 16, 16)]
                        c = d_v[r, pl.ds(k * 16, 16)]
                        d = f - c
                        sq = sq + d * d
                        d_v[r, pl.ds(k * 16, 16)] = d * SCALE
                    return sq
                sq = lax.fori_loop(0, 128, row_body, sq)
            else:
                def row_body(r, carry):
                    for k in range(4):
                        f = f_v[r, pl.ds(k * 16, 16)]
                        c = d_v[r, pl.ds(k * 16, 16)]
                        d_v[r, pl.ds(k * 16, 16)] = (f - c) * SCALE
                    return carry
                lax.fori_loop(0, 128, row_body, 0)

            pltpu.async_copy(d_v, acc_sh.at[idx_v.at[j]], sem, add=True).wait()

        plsc.subcore_barrier()

        @pl.when(full)
        def _(row0=row0):
            pltpu.sync_copy(acc_sh.at[pl.ds(sid * CHUNK, CHUNK)],
                            out_hbm.at[pl.ds(row0, CHUNK)])

        @pl.when(tail)
        def _(row0=row0):
            pltpu.sync_copy(acc_sh.at[pl.ds(sid * CHUNK, TAIL)],
                            out_hbm.at[pl.ds(row0, TAIL)])

        plsc.subcore_barrier()

    part_v[...] = sq

    @pl.when(cid == 0)
    def _():
        pltpu.sync_copy(part_v, loss_hbm.at[sid])


_sc_call = functools.partial(
    pl.kernel,
    out_type=(jax.ShapeDtypeStruct((C, D), jnp.float32),
              jax.ShapeDtypeStruct((NS, 16), jnp.float32)),
    mesh=plsc.VectorSubcoreMesh(core_axis_name="c", subcore_axis_name="s",
                                num_cores=NC, num_subcores=NS),
    scratch_types=[
        pltpu.VMEM((128, D), jnp.float32),
        pltpu.VMEM((128, D), jnp.float32),
        pltpu.VMEM((NCH, 128), jnp.int32),
        pltpu.VMEM((NCH, 128), jnp.int32),
        pltpu.VMEM((16,), jnp.float32),
        pltpu.VMEM_SHARED((RANGE + NTRASH, D), jnp.float32),
        pltpu.SemaphoreType.DMA,
    ],
    compiler_params=pltpu.CompilerParams(use_tc_tiling_on_sc=False),
)(_body)


def kernel(features, labels, center_var):
    labels = labels.reshape(-1)
    new_center, parts = _sc_call(features, labels, center_var)
    loss = jnp.sum(parts) * (1.0 / (B * D))
    return loss, new_center


# speedup vs baseline: 1.5976x; 1.5731x over previous
import functools

import jax
import jax.numpy as jnp
from jax import lax
from jax.experimental import pallas as pl
from jax.experimental.pallas import tpu as pltpu
from jax.experimental.pallas import tpu_sc as plsc

B = 16384
D = 64
C = 100000
SCALE = 0.05

NC = 2
NS = 16
PASSES = D // (NC * NS)


def _body(ctr_t, feat_t, lab_hbm, out_t, loss_hbm,
          acc_v, f_v, lab_v, part_v, lab_sh, sem):
    cid = lax.axis_index("c")
    sid = lax.axis_index("s")

    for j in range(8):
        r = sid * 8 + j
        pltpu.sync_copy(lab_hbm.at[pl.ds(r * 128, 128)], lab_sh.at[r])
    plsc.subcore_barrier()

    iota = lax.iota(jnp.int32, 16)
    sq = jnp.zeros((16,), jnp.float32)
    for p in range(PASSES):
        d = cid * (PASSES * NS) + p * NS + sid
        pltpu.sync_copy(ctr_t.at[d], acc_v)
        pltpu.sync_copy(feat_t.at[d], f_v)
        for ch in range(8):
            pltpu.sync_copy(lab_sh.at[pl.ds(ch * 16, 16)], lab_v)

            def grp_a(g, sq, ch=ch):
                r = g >> 3
                cc = (g & 7) * 16
                labv = lab_v[r, pl.ds(cc, 16)]
                s0 = ch * 2048 + g * 16
                f16 = f_v[pl.ds(s0, 16)]
                c16 = plsc.load_gather(acc_v, [labv])
                d16 = f16 - c16
                sq = sq + d16 * d16
                f_v[pl.ds(s0, 16)] = d16 * SCALE
                return sq

            sq = lax.fori_loop(0, 128, grp_a, sq)

        for ch in range(8):
            pltpu.sync_copy(lab_sh.at[pl.ds(ch * 16, 16)], lab_v)

            def grp_b(g, carry, ch=ch):
                r = g >> 3
                cc = (g & 7) * 16
                labv = lab_v[r, pl.ds(cc, 16)]
                delta = f_v[pl.ds(ch * 2048 + g * 16, 16)]
                cnt, _ = plsc.scan_count(labv)
                hasdup = jnp.max(cnt) != jnp.min(cnt)

                @pl.when(jnp.logical_not(hasdup))
                def _():
                    plsc.addupdate_scatter(acc_v, [labv], delta)

                @pl.when(hasdup)
                def _():
                    for j in range(16):
                        plsc.addupdate_scatter(acc_v, [labv], delta,
                                               mask=iota == j)
                return carry

            lax.fori_loop(0, 128, grp_b, 0)
        pltpu.sync_copy(acc_v, out_t.at[d])
    part_v[...] = sq
    wid = cid * NS + sid
    pltpu.sync_copy(part_v, loss_hbm.at[wid])


_sc_call = functools.partial(
    pl.kernel,
    out_type=(jax.ShapeDtypeStruct((D, C), jnp.float32),
              jax.ShapeDtypeStruct((NC * NS, 16), jnp.float32)),
    mesh=plsc.VectorSubcoreMesh(core_axis_name="c", subcore_axis_name="s",
                                num_cores=NC, num_subcores=NS),
    scratch_types=[
        pltpu.VMEM((C,), jnp.float32),
        pltpu.VMEM((B,), jnp.float32),
        pltpu.VMEM((16, 128), jnp.int32),
        pltpu.VMEM((16,), jnp.float32),
        pltpu.VMEM_SHARED((128, 128), jnp.int32),
        pltpu.SemaphoreType.DMA,
    ],
    compiler_params=pltpu.CompilerParams(needs_layout_passes=False),
)(_body)


def kernel(features, labels, center_var):
    labels = labels.reshape(-1)
    out_t, parts = _sc_call(center_var.T, features.T, labels)
    loss = jnp.sum(parts) * (1.0 / (B * D))
    return loss, out_t.T


# speedup vs baseline: 2.2836x; 1.4294x over previous
import functools

import jax
import jax.numpy as jnp
from jax import lax
from jax.experimental import pallas as pl
from jax.experimental.pallas import tpu as pltpu
from jax.experimental.pallas import tpu_sc as plsc

B = 16384
D = 64
C = 100000
SCALE = 0.05

NC = 2
NS = 16
PASSES = D // (NC * NS)


def _body(ctr_t, feat_t, lab_hbm, out_t, loss_hbm,
          acc_v, f_v, lab_v, part_v, lab_sh, sem):
    cid = lax.axis_index("c")
    sid = lax.axis_index("s")

    for j in range(8):
        r = sid * 8 + j
        pltpu.sync_copy(lab_hbm.at[pl.ds(r * 128, 128)], lab_sh.at[r])
    plsc.subcore_barrier()

    iota = lax.iota(jnp.int32, 16)
    sq = jnp.zeros((16,), jnp.float32)
    for p in range(PASSES):
        d = cid * (PASSES * NS) + p * NS + sid
        cp_a = pltpu.async_copy(ctr_t.at[d], acc_v, sem)
        cp_f = pltpu.async_copy(feat_t.at[d], f_v, sem)
        cp_a.wait()
        cp_f.wait()
        for ch in range(8):
            pltpu.sync_copy(lab_sh.at[pl.ds(ch * 16, 16)], lab_v)

            def grp_a(q, sq, ch=ch):
                for u in range(4):
                    g = q * 4 + u
                    labv = lab_v[g >> 3, pl.ds((g & 7) * 16, 16)]
                    s0 = ch * 2048 + g * 16
                    f16 = f_v[pl.ds(s0, 16)]
                    c16 = plsc.load_gather(acc_v, [labv])
                    d16 = f16 - c16
                    sq = sq + d16 * d16
                    f_v[pl.ds(s0, 16)] = d16 * SCALE
                return sq

            sq = lax.fori_loop(0, 32, grp_a, sq)

        for ch in range(8):
            pltpu.sync_copy(lab_sh.at[pl.ds(ch * 16, 16)], lab_v)

            def grp_b(q, carry, ch=ch):
                labvs, deltas, dups = [], [], []
                for u in range(4):
                    g = q * 4 + u
                    labv = lab_v[g >> 3, pl.ds((g & 7) * 16, 16)]
                    delta = f_v[pl.ds(ch * 2048 + g * 16, 16)]
                    cnt, _ = plsc.scan_count(labv)
                    labvs.append(labv)
                    deltas.append(delta)
                    dups.append(jnp.max(cnt) != jnp.min(cnt))
                anydup = dups[0]
                for u in range(1, 4):
                    anydup = jnp.logical_or(anydup, dups[u])

                @pl.when(jnp.logical_not(anydup))
                def _():
                    for u in range(4):
                        plsc.addupdate_scatter(acc_v, [labvs[u]], deltas[u])

                @pl.when(anydup)
                def _():
                    for u in range(4):
                        @pl.when(jnp.logical_not(dups[u]))
                        def _(u=u):
                            plsc.addupdate_scatter(acc_v, [labvs[u]],
                                                   deltas[u])

                        @pl.when(dups[u])
                        def _(u=u):
                            for j in range(16):
                                plsc.addupdate_scatter(acc_v, [labvs[u]],
                                                       deltas[u],
                                                       mask=iota == j)
                return carry

            lax.fori_loop(0, 32, grp_b, 0)
        pltpu.sync_copy(acc_v, out_t.at[d])
    part_v[...] = sq
    wid = cid * NS + sid
    pltpu.sync_copy(part_v, loss_hbm.at[wid])


_sc_call = functools.partial(
    pl.kernel,
    out_type=(jax.ShapeDtypeStruct((D, C), jnp.float32),
              jax.ShapeDtypeStruct((NC * NS, 16), jnp.float32)),
    mesh=plsc.VectorSubcoreMesh(core_axis_name="c", subcore_axis_name="s",
                                num_cores=NC, num_subcores=NS),
    scratch_types=[
        pltpu.VMEM((C,), jnp.float32),
        pltpu.VMEM((B,), jnp.float32),
        pltpu.VMEM((16, 128), jnp.int32),
        pltpu.VMEM((16,), jnp.float32),
        pltpu.VMEM_SHARED((128, 128), jnp.int32),
        pltpu.SemaphoreType.DMA,
    ],
    compiler_params=pltpu.CompilerParams(needs_layout_passes=False),
)(_body)


def kernel(features, labels, center_var):
    labels = labels.reshape(-1)
    out_t, parts = _sc_call(center_var.T, features.T, labels)
    loss = jnp.sum(parts) * (1.0 / (B * D))
    return loss, out_t.T
